# Initial kernel scaffold; baseline (speedup 1.0000x reference)
import jax
import jax.numpy as jnp
from jax.experimental import pallas as pl


def kernel(x, edge_index, batch, head_idx, tail_idx, rel_ids, params):
    raise NotImplementedError("write your pallas kernel here")


# speedup vs baseline: 4.4753x; 4.4753x over previous
import functools

import jax
import jax.numpy as jnp
from jax import lax
from jax.experimental import pallas as pl
from jax.experimental.pallas import tpu as pltpu
from jax.experimental.pallas import tpu_sc as plsc

_N = 10000
_E = 320000
_B = 64
_HID = 128
_IN0 = 64
_RELP = 256
_ATT = 256
_HEADS = 4
_MAXD_INV = 1.0 / (10 + 1e-05)

_NC = 2
_NS = 16
_CH = 80
_NCHUNK = _E // (_NS * _CH)
_NHALF = _N // _NC
_ACCR = _NHALF + 8
_WROWS = 312
_ZROWS = 104


def _leaky(v):
    return jnp.maximum(v, 0.1 * v)


def _ln(v, g, b, eps=1e-5):
    m = jnp.mean(v, axis=-1, keepdims=True)
    var = jnp.mean((v - m) ** 2, axis=-1, keepdims=True)
    return (v - m) * lax.rsqrt(var + eps) * g + b


def _dot(a, b):
    return jnp.dot(a, b, preferred_element_type=jnp.float32)


def _dotg(a, b, dims):
    return lax.dot_general(a, b, (dims, ((), ())),
                           preferred_element_type=jnp.float32)


def _embed_body(x_ref, batch_ref, rid_ref, w1t_ref, b1_ref, w2t_ref, b2_ref,
                rtab_ref, rwt_ref, rb_ref, rg_ref, rbeta_ref, out_ref):
    xf = x_ref[:].astype(jnp.float32) * _MAXD_INV
    t = (xf[:, 0:1] * w1t_ref[0:1, :] + xf[:, 1:2] * w1t_ref[1:2, :]
         + b1_ref[:])
    h0 = _dot(_leaky(t), w2t_ref[:]) + b2_ref[:]

    rid = rid_ref[:]
    oh = (rid == lax.broadcasted_iota(jnp.int32, (_B, _RELP), 1)
          ).astype(jnp.float32)
    eg = _dot(oh, rtab_ref[:])
    eg = _leaky(_dot(eg, rwt_ref[:]) + rb_ref[:])
    eg = _ln(eg, rg_ref[:], rbeta_ref[:])

    pn = (batch_ref[:] == lax.broadcasted_iota(jnp.int32, (_N, _B), 1)
          ).astype(jnp.float32)
    e = _dot(pn, eg)
    out_ref[:] = jnp.concatenate(
        [h0, e, jnp.zeros((_N, _HID - _IN0), jnp.float32)], axis=-1)


def _embed_call(x, batch2, rid2, *weights):
    return pl.pallas_call(
        _embed_body,
        out_shape=jax.ShapeDtypeStruct((_N, _HID), jnp.float32),
    )(x, batch2, rid2, *weights)


@functools.lru_cache(maxsize=None)
def _make_segsum(d):
    mesh = plsc.VectorSubcoreMesh(core_axis_name="c", subcore_axis_name="s")

    @functools.partial(
        pl.kernel,
        out_type=jax.ShapeDtypeStruct((_NC, _NHALF, d), jnp.float32),
        mesh=mesh,
        scratch_types=[
            pltpu.VMEM((_NCHUNK, _CH), jnp.int32),
            pltpu.VMEM((_NCHUNK, _CH), jnp.int32),
            pltpu.VMEM((_CH, d), jnp.float32),
            pltpu.VMEM((_ZROWS, d), jnp.float32),
            pltpu.VMEM_SHARED((_ACCR, d), jnp.float32),
            pltpu.SemaphoreType.DMA,
        ],
    )
    def seg(h_hbm, src_hbm, dst_hbm, out_hbm, src_v, dst_v, rows_v, z_v,
            acc, sem):
        c = lax.axis_index("c")
        s = lax.axis_index("s")
        base = s * _WROWS

        def zrow(r, carry):
            for k in range(d // 16):
                z_v[r, pl.ds(k * 16, 16)] = jnp.zeros((16,), jnp.float32)
            return carry
        lax.fori_loop(0, _ZROWS, zrow, 0)
        for k in range(_WROWS // _ZROWS):
            pltpu.sync_copy(z_v, acc.at[pl.ds(base + k * _ZROWS, _ZROWS)])

        @pl.when(s == 0)
        def _zero_tail():
            pltpu.sync_copy(z_v.at[pl.ds(0, 16)],
                            acc.at[pl.ds(_NS * _WROWS, _ACCR - _NS * _WROWS)])

        pltpu.sync_copy(src_hbm.at[s], src_v)
        pltpu.sync_copy(dst_hbm.at[c, s], dst_v)
        plsc.subcore_barrier()

        def chunk(j, carry):
            pltpu.async_copy(h_hbm.at[src_v.at[j]], rows_v, sem).wait()
            pltpu.sync_copy(rows_v, acc.at[dst_v.at[j]], add=True)
            return carry
        lax.fori_loop(0, _NCHUNK, chunk, 0)

        plsc.subcore_barrier()
        pltpu.sync_copy(acc.at[pl.ds(base, _WROWS)],
                        out_hbm.at[c, pl.ds(base, _WROWS)])

        @pl.when(s == 0)
        def _write_tail():
            pltpu.sync_copy(acc.at[pl.ds(_NS * _WROWS, _NHALF - _NS * _WROWS)],
                            out_hbm.at[c, pl.ds(_NS * _WROWS,
                                                _NHALF - _NS * _WROWS)])

    return seg


def _layer_body(a_ref, h_ref, wrelt_ref, wnodet_ref, bn_ref, sg_ref, bb_ref,
                lg_ref, lb_ref, out_ref):
    out = (_dot(a_ref[:], wrelt_ref[:]) + _dot(h_ref[:], wnodet_ref[:])
           + bn_ref[:])
    out = out * sg_ref[:] + bb_ref[:]
    out = _leaky(out)
    out = _ln(out, lg_ref[:], lb_ref[:])
    out_ref[:] = _leaky(out)


def _layer_call(a, h, *weights):
    return pl.pallas_call(
        _layer_body,
        out_shape=jax.ShapeDtypeStruct((_N, _HID), jnp.float32),
    )(a, h, *weights)


def _pool_body(a_ref, h_ref, wrelt_ref, wnodet_ref, bn_ref, sg_ref, bb_ref,
               lg_ref, lb_ref, batch_ref, hidx_ref, tidx_ref, apwt_ref,
               q_ref, apg_ref, apb_ref, w1t_ref, b1_ref, w2t_ref, b2_ref,
               w3t_ref, b3_ref, out_ref):
    out = (_dot(a_ref[:], wrelt_ref[:]) + _dot(h_ref[:], wnodet_ref[:])
           + bn_ref[:])
    out = out * sg_ref[:] + bb_ref[:]
    h = _ln(_leaky(out), lg_ref[:], lb_ref[:])

    keys = _dot(h, apwt_ref[:])
    prod = keys * q_ref[:]
    hsel = ((lax.broadcasted_iota(jnp.int32, (_ATT, _HEADS), 0) // 64)
            == lax.broadcasted_iota(jnp.int32, (_ATT, _HEADS), 1)
            ).astype(jnp.float32)
    att = _dot(prod, hsel)

    pn = (batch_ref[:] == lax.broadcasted_iota(jnp.int32, (_N, _B), 1)
          ).astype(jnp.float32)
    maxes = []
    for hd in range(_HEADS):
        m = jnp.where(pn > 0.0, att[:, hd:hd + 1], -1e30)
        maxes.append(jnp.max(m, axis=0, keepdims=True))
    amax = jnp.concatenate(maxes, axis=0)
    amax_n = _dotg(pn, amax, (((1,), (1,))))
    ex = jnp.exp(att - amax_n)
    denom = _dotg(pn, ex, (((0,), (0,))))
    exb = _dotg(ex, hsel, (((1,), (1,))))
    u = _dotg(pn, exb * keys, (((0,), (0,))))
    den256 = _dotg(denom, hsel, (((1,), (1,))))
    den256 = jnp.where(den256 == 0.0, 1.0, den256)
    zg = _ln(u / den256, apg_ref[:], apb_ref[:])

    io_n = lax.broadcasted_iota(jnp.int32, (_B, _N), 1)
    ph = (hidx_ref[:] == io_n).astype(jnp.float32)
    pt = (tidx_ref[:] == io_n).astype(jnp.float32)
    hr = _dot(ph, h)
    tr = _dot(pt, h)
    feats = jnp.concatenate([zg, hr, tr], axis=-1)
    s1 = _leaky(_dot(feats, w1t_ref[:]) + b1_ref[:])
    s2 = _leaky(_dot(s1, w2t_ref[:]) + b2_ref[:])
    out_ref[:] = _dot(s2, w3t_ref[:]) + b3_ref[:]


def _pool_call(a, h, *rest):
    return pl.pallas_call(
        _pool_body,
        out_shape=jax.ShapeDtypeStruct((_B, 1), jnp.float32),
    )(a, h, *rest)


def kernel(x, edge_index, batch, head_idx, tail_idx, rel_ids, params):
    p = params
    f32 = jnp.float32
    row = lambda v: v.astype(f32).reshape(1, -1)

    batch2 = batch.astype(jnp.int32).reshape(_N, 1)
    rid2 = rel_ids.astype(jnp.int32).reshape(_B, 1)
    hidx2 = head_idx.astype(jnp.int32).reshape(_B, 1)
    tidx2 = tail_idx.astype(jnp.int32).reshape(_B, 1)
    src3 = edge_index[0].astype(jnp.int32).reshape(_NS, _NCHUNK, _CH)
    dst = edge_index[1].astype(jnp.int32)
    dst4 = jnp.stack([
        jnp.where((dst >= c * _NHALF) & (dst < (c + 1) * _NHALF),
                  dst - c * _NHALF, _NHALF)
        for c in range(_NC)]).reshape(_NC, _NS, _NCHUNK, _CH)

    rtab = jnp.pad(p['rel_table'].astype(f32),
                   ((0, _RELP - p['rel_table'].shape[0]), (0, 0)))

    h = _embed_call(
        x.astype(jnp.int32), batch2, rid2,
        p['ne_W1'].T.astype(f32), row(p['ne_b1']),
        p['ne_W2'].T.astype(f32), row(p['ne_b2']),
        rtab, p['rel_W'].T.astype(f32), row(p['rel_b']),
        row(p['rel_g']), row(p['rel_beta']))

    bn_scale = 1.0 / jnp.sqrt(jnp.float32(1.0 + 1e-05))
    seg = _make_segsum(_HID)
    for i in range(3):
        a = seg(h, src3, dst4).reshape(_N, _HID)
        wrel_t = p['l%d_wrel' % i].T.astype(f32)
        wnode_t = p['l%d_wnode' % i].T.astype(f32)
        if i == 0:
            wrel_t = jnp.pad(wrel_t, ((0, _HID - _IN0), (0, 0)))
            wnode_t = jnp.pad(wnode_t, ((0, _HID - _IN0), (0, 0)))
        lw = (wrel_t, wnode_t,
              row(p['l%d_bnode' % i]),
              row(p['l%d_bng' % i]) * bn_scale,
              row(p['l%d_bnb' % i]),
              row(p['n%d_g' % i]), row(p['n%d_b' % i]))
        if i < 2:
            h = _layer_call(a, h, *lw)
        else:
            score = _pool_call(
                a, h, *lw, batch2, hidx2, tidx2,
                p['ap_W'].T.astype(f32), p['ap_q'].reshape(1, _ATT).astype(f32),
                row(p['ap_g']), row(p['ap_b']),
                p['sc_W1'].T.astype(f32), row(p['sc_b1']),
                p['sc_W2'].T.astype(f32), row(p['sc_b2']),
                p['sc_W3'].T.astype(f32), row(p['sc_b3']))
    return score.reshape(_B)
